# Initial kernel scaffold; baseline (speedup 1.0000x reference)
import jax
import jax.numpy as jnp
from jax.experimental import pallas as pl


def kernel(features, pts_src, pts_dst, invis_idx, height, width):
    raise NotImplementedError("write your pallas kernel here")


# speedup vs baseline: 1.8174x; 1.8174x over previous
import functools

import jax
import jax.numpy as jnp
from jax.experimental import pallas as pl
from jax.experimental.pallas import tpu as pltpu


def _normalize_body(f_ref, u_ref):
    f = f_ref[0]
    ssq = jnp.sum(f * f, axis=-1, keepdims=True)
    inv = jax.lax.rsqrt(jnp.maximum(ssq, 1e-24))
    u_ref[0] = (f * inv).astype(u_ref.dtype)


def _pair_body(ua_ref, ub_ref, sp_ref, dt_ref, cnt_ref, tot_ref):
    g = pl.program_id(0)

    @pl.when(g == 0)
    def _init():
        cnt_ref[0, 0] = 0.0
        tot_ref[0, 0] = 0.0

    sp = sp_ref[0]
    xs = sp[:, 0:1]
    ys = sp[:, 1:2]
    dt = dt_ref[0]
    xd = dt[0:1, :]
    yd = dt[1:2, :]

    a2 = xs * xs + ys * ys
    b2 = xd * xd + yd * yd
    d2 = (a2 + b2) - 2.0 * (xs * xd + ys * yd)
    maskf = (d2 <= 1.0).astype(jnp.float32)

    cos = jax.lax.dot_general(
        ua_ref[0], ub_ref[0],
        dimension_numbers=(((1,), (1,)), ((), ())),
        preferred_element_type=jnp.float32)

    cnt = jnp.sum(maskf)
    mcos = jnp.sum(maskf * cos)
    cnt_ref[0, 0] += cnt
    tot_ref[0, 0] += cnt - mcos


def kernel(features, pts_src, pts_dst, invis_idx, height, width):
    del invis_idx
    B, N, D = features.shape
    radius = 1.0
    fx = (jnp.asarray(width, jnp.float32) - 1.0) / 2.0
    fy = (jnp.asarray(height, jnp.float32) - 1.0) / 2.0
    factor = jnp.stack([fx, fy]) / radius

    src_pix = (pts_src + 1.0) * factor
    dst_pix = (pts_dst.reshape(B * B, N, 2) + 1.0) * factor

    src_p = jnp.pad(src_pix, ((0, 0), (0, 0), (0, 6)))
    dst_t = jnp.pad(jnp.transpose(dst_pix, (0, 2, 1)),
                    ((0, 0), (0, 6), (0, 0)))

    u = pl.pallas_call(
        _normalize_body,
        grid=(B,),
        in_specs=[pl.BlockSpec((1, N, D), lambda b: (b, 0, 0))],
        out_specs=pl.BlockSpec((1, N, D), lambda b: (b, 0, 0)),
        out_shape=jax.ShapeDtypeStruct((B, N, D), jnp.bfloat16),
    )(features)

    cnt, tot = pl.pallas_call(
        _pair_body,
        grid=(B * B,),
        in_specs=[
            pl.BlockSpec((1, N, D), lambda g: (g % B, 0, 0)),
            pl.BlockSpec((1, N, D), lambda g: (g // B, 0, 0)),
            pl.BlockSpec((1, N, 8), lambda g: (g // B, 0, 0)),
            pl.BlockSpec((1, 8, N), lambda g: (g, 0, 0)),
        ],
        out_specs=[
            pl.BlockSpec(memory_space=pltpu.SMEM),
            pl.BlockSpec(memory_space=pltpu.SMEM),
        ],
        out_shape=[
            jax.ShapeDtypeStruct((1, 1), jnp.float32),
            jax.ShapeDtypeStruct((1, 1), jnp.float32),
        ],
        compiler_params=pltpu.CompilerParams(
            dimension_semantics=("arbitrary",)),
    )(u, u, src_p, dst_t)

    return tot[0, 0] / cnt[0, 0]
